# baseline (speedup 1.0000x reference)
import functools

import jax
import jax.numpy as jnp
from jax import lax
from jax.experimental import pallas as pl
from jax.experimental.pallas import tpu as pltpu
from jax.experimental.pallas import tpu_sc as plsc

N_USERS = 5000
N = 10000
E = 160000
D = 256
DH = 128
NW = 32
NSUB = 16
EPW = E // NW
CB = 40
NCHB = EPW // CB
NDEEP = 3
RPS = N // NSUB
TB = 1000


def _tx_body(with_r, *refs):
    x_ref, vi_ref, vj_ref, ui_ref, uj_ref = refs[:5]
    k = 5
    r_ref = refs[k] if with_r else None
    k += 1 if with_r else 0
    bv_ref, bu_ref = refs[k], refs[k + 1]
    outs = refs[k + 2:]
    vix0_o, vix1_o, pju0_o, pju1_o, ujx_o = outs[:5]
    x = x_ref[...]
    vix = jnp.dot(x, vi_ref[...], preferred_element_type=jnp.float32) + bv_ref[...]
    vjx = jnp.dot(x, vj_ref[...], preferred_element_type=jnp.float32)
    uix = jnp.dot(x, ui_ref[...], preferred_element_type=jnp.float32)
    ujx_o[...] = jnp.dot(x, uj_ref[...], preferred_element_type=jnp.float32) + bu_ref[...]
    vix0_o[...] = vix[:, :DH]
    vix1_o[...] = vix[:, DH:]
    pju0_o[...] = jnp.concatenate([vjx[:, :DH], uix[:, :DH]], axis=1)
    pju1_o[...] = jnp.concatenate([vjx[:, DH:], uix[:, DH:]], axis=1)
    if with_r:
        outs[5][...] = jnp.dot(x, r_ref[...], preferred_element_type=jnp.float32)


def _transforms(x, Vi, Vj, Ui, Uj, bv, bu, R=None):
    with_r = R is not None
    grid = (N // TB,)
    row = pl.BlockSpec((TB, D), lambda i: (i, 0))
    half = pl.BlockSpec((TB, DH), lambda i: (i, 0))
    w = pl.BlockSpec((D, D), lambda i: (0, 0))
    b = pl.BlockSpec((1, D), lambda i: (0, 0))
    in_specs = [row, w, w, w, w] + ([w] if with_r else []) + [b, b]
    out_specs = [half, half, row, row, row] + ([row] if with_r else [])
    out_shape = ([jax.ShapeDtypeStruct((N, DH), jnp.float32)] * 2
                 + [jax.ShapeDtypeStruct((N, D), jnp.float32)] * 3
                 + ([jax.ShapeDtypeStruct((N, D), jnp.float32)] if with_r else []))
    args = (x, Vi, Vj, Ui, Uj) + ((R,) if with_r else ()) + (bv, bu)
    return pl.pallas_call(
        functools.partial(_tx_body, with_r),
        grid=grid, in_specs=in_specs, out_specs=out_specs, out_shape=out_shape,
    )(*args)


def _bn_body(with_rx, relu_out, *refs):
    if with_rx:
        ujx_ref, p0_ref, p1_ref, rx_ref, out_ref, acc_ref = refs
    else:
        ujx_ref, p0_ref, p1_ref, out_ref, acc_ref = refs
        rx_ref = None
    ph = pl.program_id(0)
    i = pl.program_id(1)
    h = ujx_ref[...] + jnp.concatenate(
        [p0_ref[0] + p0_ref[1], p1_ref[0] + p1_ref[1]], axis=1)

    @pl.when((ph == 0) & (i == 0))
    def _():
        acc_ref[...] = jnp.zeros_like(acc_ref)

    @pl.when(ph == 0)
    def _():
        acc_ref[0:1] += jnp.sum(h, axis=0, keepdims=True)
        acc_ref[1:2] += jnp.sum(h * h, axis=0, keepdims=True)

    @pl.when(ph == 1)
    def _():
        m = acc_ref[0:1] / N
        v = acc_ref[1:2] / N - m * m
        o = (h - m) * lax.rsqrt(v + 1e-3)
        if with_rx:
            o = o + rx_ref[...]
        if relu_out:
            o = jnp.maximum(o, 0.0)
        out_ref[...] = o


def _bn_stage(ujx, p0, p1, rx=None, relu_out=True):
    with_rx = rx is not None
    row = pl.BlockSpec((TB, D), lambda p, i: (i, 0))
    part = pl.BlockSpec((2, TB, DH), lambda p, i: (0, i, 0))
    in_specs = [row, part, part] + ([row] if with_rx else [])
    args = (ujx, p0, p1) + ((rx,) if with_rx else ())
    return pl.pallas_call(
        functools.partial(_bn_body, with_rx, relu_out),
        grid=(2, N // TB),
        in_specs=in_specs,
        out_specs=row,
        out_shape=jax.ShapeDtypeStruct((N, D), jnp.float32),
        scratch_shapes=[pltpu.VMEM((8, D), jnp.float32)],
    )(*args)


def _sc_agg(vix0, vix1, pju0, pju1, se0, se1):
    mesh = plsc.VectorSubcoreMesh(core_axis_name="c", subcore_axis_name="s")
    out_type = [jax.ShapeDtypeStruct((NW, RPS, DH), jnp.float32)] * 2
    scratch = (
        [pltpu.VMEM((2, CB), jnp.int32)] * NDEEP
        + [pltpu.VMEM((CB, DH), jnp.float32)] * NDEEP
        + [pltpu.VMEM((CB, 2 * DH), jnp.float32)] * NDEEP
        + [pltpu.VMEM_SHARED((N, DH), jnp.float32)]
        + [pltpu.SemaphoreType.DMA] * (3 * NDEEP)
    )

    @functools.partial(pl.kernel, out_type=out_type, mesh=mesh,
                       scratch_types=scratch)
    def k(vix0_h, vix1_h, pju0_h, pju1_h, se0_h, se1_h,
          o0_h, o1_h, sebuf0, sebuf1, sebuf2, abuf0, abuf1, abuf2,
          bbuf0, bbuf1, bbuf2, agg,
          sa0, sa1, sa2, sb0, sb1, sb2, sc0, sc1, sc2):
        cid = lax.axis_index("c")
        sid = lax.axis_index("s")
        wid = cid * NSUB + sid
        cbase = wid * NCHB
        zeros16 = jnp.zeros((16,), jnp.float32)
        vix_t = (vix0_h, vix1_h)
        pju_t = (pju0_h, pju1_h)
        se_t = (se0_h, se1_h)
        o_t = (o0_h, o1_h)
        sets = ((sebuf0, abuf0, bbuf0, sa0, sb0, sc0),
                (sebuf1, abuf1, bbuf1, sa1, sb1, sc1),
                (sebuf2, abuf2, bbuf2, sa2, sb2, sc2))

        for p in range(2):
            vix_h, pju_h, se_h = vix_t[p], pju_t[p], se_t[p]

            def fire(j, sebuf, abuf, bbuf, sa, sb, sc):
                pltpu.sync_copy(se_h.at[cbase + j], sebuf)
                pltpu.make_async_copy(vix_h.at[sebuf.at[1]], abuf, sa).start()
                pltpu.make_async_copy(pju_h.at[sebuf.at[0]], bbuf, sb).start()

            @plsc.parallel_loop(0, CB, step=1, unroll=4)
            def _(r):
                for l in range(DH // 16):
                    abuf0[r, pl.ds(l * 16, 16)] = zeros16
            base = sid * RPS
            for z in range(RPS // CB):
                pltpu.sync_copy(abuf0, agg.at[pl.ds(base + z * CB, CB)])
            pltpu.sync_copy(abuf0.at[pl.ds(0, RPS % CB)],
                            agg.at[pl.ds(base + (RPS // CB) * CB, RPS % CB)])
            plsc.subcore_barrier()

            for b in range(NDEEP):
                fire(b, *sets[b])

            def body(j, sebuf, abuf, bbuf, sa, sb, sc):
                pltpu.make_async_copy(vix_h.at[sebuf.at[1]], abuf, sa).wait()
                pltpu.make_async_copy(pju_h.at[sebuf.at[0]], bbuf, sb).wait()

                @plsc.parallel_loop(0, CB, step=1, unroll=4)
                def _(r):
                    for l in range(DH // 16):
                        sl = pl.ds(l * 16, 16)
                        gate_in = abuf[r, sl] + bbuf[r, sl]
                        abuf[r, sl] = (bbuf[r, pl.ds(DH + l * 16, 16)]
                                       / (1.0 + jnp.exp(-gate_in)))

                scat = pltpu.async_copy(abuf, agg.at[sebuf.at[1]], sc, add=True)

                @pl.when(j + NDEEP < NCHB)
                def _():
                    pltpu.sync_copy(se_h.at[cbase + j + NDEEP], sebuf)
                scat.wait()

                @pl.when(j + NDEEP < NCHB)
                def _():
                    pltpu.make_async_copy(vix_h.at[sebuf.at[1]], abuf, sa).start()
                    pltpu.make_async_copy(pju_h.at[sebuf.at[0]], bbuf, sb).start()

            def step(g, _):
                for b in range(NDEEP):
                    j = g * NDEEP + b

                    @pl.when(j < NCHB)
                    def _():
                        body(j, *sets[b])
                return 0
            lax.fori_loop(0, (NCHB + NDEEP - 1) // NDEEP, step, 0)
            plsc.subcore_barrier()

            pltpu.sync_copy(agg.at[pl.ds(sid * RPS, RPS)], o_t[p].at[wid])
            plsc.subcore_barrier()

    return k(vix0, vix1, pju0, pju1, se0, se1)


def _mk_se(s, e):
    return jnp.stack([s.reshape(NW * NCHB, CB), e.reshape(NW * NCHB, CB)],
                     axis=1)


def kernel(users, items, start0, end0, start1, end1,
           Ui1, Uj1, Vi1, Vj1, bu1, bv1,
           Ui2, Uj2, Vi2, Vj2, bu2, bv2, R):
    x0 = jnp.concatenate([users, items], axis=0)
    se0 = _mk_se(start0, end0)
    se1 = _mk_se(start1, end1)
    bv1r = bv1.reshape(1, D)
    bu1r = bu1.reshape(1, D)

    vix0, vix1, pju0, pju1, ujx1, rx = _transforms(
        x0, Vi1, Vj1, Ui1, Uj1, bv1r, bu1r, R)
    p0, p1 = _sc_agg(vix0, vix1, pju0, pju1, se0, se1)
    o1 = _bn_stage(ujx1, p0.reshape(2, N, DH), p1.reshape(2, N, DH))

    vix0, vix1, pju0, pju1, ujx2 = _transforms(o1, Vi2, Vj2, Ui2, Uj2, bv1r, bu1r)
    q0, q1 = _sc_agg(vix0, vix1, pju0, pju1, se0, se1)
    out = _bn_stage(ujx2, q0.reshape(2, N, DH), q1.reshape(2, N, DH), rx,
                    relu_out=True)
    return out[:N_USERS], out[N_USERS:]


# speedup vs baseline: 1.3339x; 1.3339x over previous
import functools

import jax
import jax.numpy as jnp
from jax import lax
from jax.experimental import pallas as pl
from jax.experimental.pallas import tpu as pltpu
from jax.experimental.pallas import tpu_sc as plsc

N_USERS = 5000
N = 10000
E = 160000
D = 256
DH = 128
NW = 32
NSUB = 16
EPW = E // NW
CB = 64
NCHB = 80
EPAD = (NW - 1) * EPW + NCHB * CB
RPS = N // NSUB
NTRASH = 8
TB = 1000


def _tx_body(with_r, *refs):
    x_ref, vi_ref, vj_ref, ui_ref, uj_ref = refs[:5]
    k = 5
    r_ref = refs[k] if with_r else None
    k += 1 if with_r else 0
    bv_ref, bu_ref = refs[k], refs[k + 1]
    outs = refs[k + 2:]
    vix0_o, vix1_o, pju0_o, pju1_o, ujx_o = outs[:5]
    x = x_ref[...]
    vix = jnp.dot(x, vi_ref[...], preferred_element_type=jnp.float32) + bv_ref[...]
    vjx = jnp.dot(x, vj_ref[...], preferred_element_type=jnp.float32)
    uix = jnp.dot(x, ui_ref[...], preferred_element_type=jnp.float32)
    ujx_o[...] = jnp.dot(x, uj_ref[...], preferred_element_type=jnp.float32) + bu_ref[...]
    vix0_o[...] = vix[:, :DH]
    vix1_o[...] = vix[:, DH:]
    pju0_o[...] = jnp.concatenate([vjx[:, :DH], uix[:, :DH]], axis=1)
    pju1_o[...] = jnp.concatenate([vjx[:, DH:], uix[:, DH:]], axis=1)
    if with_r:
        outs[5][...] = jnp.dot(x, r_ref[...], preferred_element_type=jnp.float32)


def _transforms(x, Vi, Vj, Ui, Uj, bv, bu, R=None):
    with_r = R is not None
    grid = (N // TB,)
    row = pl.BlockSpec((TB, D), lambda i: (i, 0))
    half = pl.BlockSpec((TB, DH), lambda i: (i, 0))
    w = pl.BlockSpec((D, D), lambda i: (0, 0))
    b = pl.BlockSpec((1, D), lambda i: (0, 0))
    in_specs = [row, w, w, w, w] + ([w] if with_r else []) + [b, b]
    out_specs = [half, half, row, row, row] + ([row] if with_r else [])
    out_shape = ([jax.ShapeDtypeStruct((N, DH), jnp.float32)] * 2
                 + [jax.ShapeDtypeStruct((N, D), jnp.float32)] * 3
                 + ([jax.ShapeDtypeStruct((N, D), jnp.float32)] if with_r else []))
    args = (x, Vi, Vj, Ui, Uj) + ((R,) if with_r else ()) + (bv, bu)
    return pl.pallas_call(
        functools.partial(_tx_body, with_r),
        grid=grid, in_specs=in_specs, out_specs=out_specs, out_shape=out_shape,
    )(*args)


def _bn_body(with_rx, relu_out, *refs):
    if with_rx:
        ujx_ref, p0_ref, p1_ref, rx_ref, out_ref, acc_ref = refs
    else:
        ujx_ref, p0_ref, p1_ref, out_ref, acc_ref = refs
        rx_ref = None
    ph = pl.program_id(0)
    i = pl.program_id(1)
    h = ujx_ref[...] + jnp.concatenate(
        [p0_ref[0] + p0_ref[1], p1_ref[0] + p1_ref[1]], axis=1)

    @pl.when((ph == 0) & (i == 0))
    def _():
        acc_ref[...] = jnp.zeros_like(acc_ref)

    @pl.when(ph == 0)
    def _():
        acc_ref[0:1] += jnp.sum(h, axis=0, keepdims=True)
        acc_ref[1:2] += jnp.sum(h * h, axis=0, keepdims=True)

    @pl.when(ph == 1)
    def _():
        m = acc_ref[0:1] / N
        v = acc_ref[1:2] / N - m * m
        o = (h - m) * lax.rsqrt(v + 1e-3)
        if with_rx:
            o = o + rx_ref[...]
        if relu_out:
            o = jnp.maximum(o, 0.0)
        out_ref[...] = o


def _bn_stage(ujx, p0, p1, rx=None, relu_out=True):
    with_rx = rx is not None
    row = pl.BlockSpec((TB, D), lambda p, i: (i, 0))
    part = pl.BlockSpec((2, TB, DH), lambda p, i: (0, i, 0))
    in_specs = [row, part, part] + ([row] if with_rx else [])
    args = (ujx, p0, p1) + ((rx,) if with_rx else ())
    return pl.pallas_call(
        functools.partial(_bn_body, with_rx, relu_out),
        grid=(2, N // TB),
        in_specs=in_specs,
        out_specs=row,
        out_shape=jax.ShapeDtypeStruct((N, D), jnp.float32),
        scratch_shapes=[pltpu.VMEM((8, D), jnp.float32)],
    )(*args)


def _sc_agg(vix0, vix1, pju0, pju1, se0, se1):
    mesh = plsc.VectorSubcoreMesh(core_axis_name="c", subcore_axis_name="s")
    out_type = [jax.ShapeDtypeStruct((NW, RPS, DH), jnp.float32)] * 2
    scratch = [
        pltpu.VMEM((2, CB), jnp.int32), pltpu.VMEM((2, CB), jnp.int32),
        pltpu.VMEM((CB,), jnp.int32), pltpu.VMEM((CB,), jnp.int32),
        pltpu.VMEM((CB, DH), jnp.float32), pltpu.VMEM((CB, DH), jnp.float32),
        pltpu.VMEM((CB, 2 * DH), jnp.float32), pltpu.VMEM((CB, 2 * DH), jnp.float32),
        pltpu.VMEM_SHARED((N + NTRASH, DH), jnp.float32),
        pltpu.SemaphoreType.DMA, pltpu.SemaphoreType.DMA,
        pltpu.SemaphoreType.DMA, pltpu.SemaphoreType.DMA,
        pltpu.SemaphoreType.DMA, pltpu.SemaphoreType.DMA,
        pltpu.SemaphoreType.DMA, pltpu.SemaphoreType.DMA,
    ]

    @functools.partial(pl.kernel, out_type=out_type, mesh=mesh,
                       scratch_types=scratch)
    def k(vix0_h, vix1_h, pju0_h, pju1_h, se0_h, se1_h,
          o0_h, o1_h, sebuf0, sebuf1, scat0, scat1, abuf0, abuf1,
          bbuf0, bbuf1, agg, sa0, sa1, sb0, sb1, sc0, sc1, si0, si1):
        cid = lax.axis_index("c")
        sid = lax.axis_index("s")
        wid = cid * NSUB + sid
        cbase = wid * NCHB
        zeros16 = jnp.zeros((16,), jnp.float32)
        lane = lax.iota(jnp.int32, 16)
        vix_t = (vix0_h, vix1_h)
        pju_t = (pju0_h, pju1_h)
        se_t = (se0_h, se1_h)
        o_t = (o0_h, o1_h)
        sets = ((sebuf0, scat0, abuf0, bbuf0, sa0, sb0, sc0, si0),
                (sebuf1, scat1, abuf1, bbuf1, sa1, sb1, sc1, si1))

        for p in range(2):
            vix_h, pju_h, se_h = vix_t[p], pju_t[p], se_t[p]

            def fire(j, sebuf, scatbuf, abuf, bbuf, sa, sb, sc, si):
                pltpu.sync_copy(se_h.at[cbase + j], sebuf)
                pltpu.make_async_copy(vix_h.at[sebuf.at[1]], abuf, sa).start()
                pltpu.make_async_copy(pju_h.at[sebuf.at[0]], bbuf, sb).start()

            @plsc.parallel_loop(0, CB, step=1, unroll=4)
            def _(r):
                for l in range(DH // 16):
                    abuf0[r, pl.ds(l * 16, 16)] = zeros16
            base = sid * RPS
            for z in range(RPS // CB):
                pltpu.sync_copy(abuf0, agg.at[pl.ds(base + z * CB, CB)])
            pltpu.sync_copy(abuf0.at[pl.ds(0, RPS % CB)],
                            agg.at[pl.ds(base + (RPS // CB) * CB, RPS % CB)])
            plsc.subcore_barrier()

            for b in range(2):
                fire(b, *sets[b])

            def step(g, _):
                for b in range(2):
                    j = g * 2 + b
                    sebuf, scatbuf, abuf, bbuf, sa, sb, sc, si = sets[b]
                    pltpu.make_async_copy(vix_h.at[sebuf.at[1]], abuf, sa).wait()
                    pltpu.make_async_copy(pju_h.at[sebuf.at[0]], bbuf, sb).wait()

                    valid = EPW - j * CB
                    for l in range(CB // 16):
                        sl = pl.ds(l * 16, 16)
                        scatbuf[sl] = jnp.where(lane + l * 16 < valid,
                                                sebuf[1, sl], N)
                    idx = pltpu.make_async_copy(se_h.at[cbase + j + 2], sebuf, si)

                    @pl.when(j + 2 < NCHB)
                    def _():
                        idx.start()

                    @plsc.parallel_loop(0, CB, step=1, unroll=4)
                    def _(r):
                        for l in range(DH // 16):
                            sl = pl.ds(l * 16, 16)
                            gate_in = abuf[r, sl] + bbuf[r, sl]
                            abuf[r, sl] = (bbuf[r, pl.ds(DH + l * 16, 16)]
                                           / (1.0 + jnp.exp(-gate_in)))

                    scat = pltpu.async_copy(abuf, agg.at[scatbuf], sc, add=True)
                    scat.wait()

                    @pl.when(j + 2 < NCHB)
                    def _():
                        idx.wait()
                        pltpu.make_async_copy(vix_h.at[sebuf.at[1]], abuf, sa).start()
                        pltpu.make_async_copy(pju_h.at[sebuf.at[0]], bbuf, sb).start()
                return 0
            lax.fori_loop(0, NCHB // 2, step, 0)
            plsc.subcore_barrier()

            pltpu.sync_copy(agg.at[pl.ds(sid * RPS, RPS)], o_t[p].at[wid])
            plsc.subcore_barrier()

    return k(vix0, vix1, pju0, pju1, se0, se1)


def _mk_se(s, e):
    pos = (jnp.arange(NW)[:, None, None] * EPW
           + jnp.arange(NCHB)[None, :, None] * CB
           + jnp.arange(CB)[None, None, :]).reshape(NW * NCHB, CB)
    sp = jnp.pad(s, (0, EPAD - E))[pos]
    ep = jnp.pad(e, (0, EPAD - E))[pos]
    return jnp.stack([sp, ep], axis=1)


def kernel(users, items, start0, end0, start1, end1,
           Ui1, Uj1, Vi1, Vj1, bu1, bv1,
           Ui2, Uj2, Vi2, Vj2, bu2, bv2, R):
    x0 = jnp.concatenate([users, items], axis=0)
    se0 = _mk_se(start0, end0)
    se1 = _mk_se(start1, end1)
    bv1r = bv1.reshape(1, D)
    bu1r = bu1.reshape(1, D)

    vix0, vix1, pju0, pju1, ujx1, rx = _transforms(
        x0, Vi1, Vj1, Ui1, Uj1, bv1r, bu1r, R)
    p0, p1 = _sc_agg(vix0, vix1, pju0, pju1, se0, se1)
    o1 = _bn_stage(ujx1, p0.reshape(2, N, DH), p1.reshape(2, N, DH))

    vix0, vix1, pju0, pju1, ujx2 = _transforms(o1, Vi2, Vj2, Ui2, Uj2, bv1r, bu1r)
    q0, q1 = _sc_agg(vix0, vix1, pju0, pju1, se0, se1)
    out = _bn_stage(ujx2, q0.reshape(2, N, DH), q1.reshape(2, N, DH), rx,
                    relu_out=True)
    return out[:N_USERS], out[N_USERS:]
